# Initial kernel scaffold; baseline (speedup 1.0000x reference)
import jax
import jax.numpy as jnp
from jax.experimental import pallas as pl


def kernel(coords, edge_index, edge_features, W_in, b_in, W_msg, b_msg, W_upd, b_upd, W_out, b_out):
    raise NotImplementedError("write your pallas kernel here")


# speedup vs baseline: 3.1862x; 3.1862x over previous
import jax
import jax.numpy as jnp
from jax import lax
from jax.experimental import pallas as pl
from jax.experimental.pallas import tpu as pltpu
from jax.experimental.pallas import tpu_sc as plsc

N = 10000
E = 320000
D = 128
DE = 16
NL = 21
TEMP = 0.1

NC = 2
NS = 16
NW = NC * NS
ET = E // NW
K = 80
NCH = ET // K
ZCH = N // K

LANES = 16
NB = 1000
EB = 2000


def _tc_nodes(coords_ref, win_ref, bin_ref, w1_ref, w2_ref, bmsg_ref,
              h_ref, a_ref, b_ref):
    h = jnp.maximum(
        jnp.dot(coords_ref[...], win_ref[...],
                preferred_element_type=jnp.float32) + bin_ref[...], 0.0)
    h_ref[...] = h
    a_ref[...] = jnp.dot(h, w1_ref[...], preferred_element_type=jnp.float32)
    b_ref[...] = (jnp.dot(h, w2_ref[...], preferred_element_type=jnp.float32)
                  + bmsg_ref[...])


def _tc_edgefeat(ef_ref, w3_ref, c_ref):
    c_ref[...] = jnp.dot(ef_ref[...], w3_ref[...],
                         preferred_element_type=jnp.float32)


def _tc_out(h_ref, a0_ref, a1_ref, u1_ref, u2_ref, bupd_ref,
            wout_ref, bout_ref, out_ref):
    agg = a0_ref[...] + a1_ref[...]
    h2 = jnp.maximum(
        jnp.dot(h_ref[...], u1_ref[...], preferred_element_type=jnp.float32)
        + jnp.dot(agg, u2_ref[...], preferred_element_type=jnp.float32)
        + bupd_ref[...], 0.0)
    out_ref[...] = (jnp.dot(h2, wout_ref[...],
                            preferred_element_type=jnp.float32)
                    + bout_ref[...]) * (1.0 / TEMP)


def _sc_edges(a_hbm, b_hbm, c_hbm, src_hbm, dst_hbm, out_hbm,
              srcv, dstv, bufa, bufb, bufc, aggs, sema, semb, semc):
    cid = lax.axis_index("c")
    sid = lax.axis_index("s")
    wid = cid * NS + sid

    zero16 = jnp.zeros((LANES,), jnp.float32)

    def zfill(i, carry):
        for j in range(D // LANES):
            bufc[i, pl.ds(j * LANES, LANES)] = zero16
        return carry

    lax.fori_loop(0, K, zfill, 0)
    for t in range((ZCH + NS - 1) // NS):
        c = sid + NS * t
        @pl.when(c < ZCH)
        def _():
            pltpu.sync_copy(bufc, aggs.at[pl.ds(c * K, K)])
    plsc.subcore_barrier()

    def chunk(ch, carry):
        base = wid * ET + ch * K
        pltpu.sync_copy(src_hbm.at[pl.ds(base, K)], srcv)
        pltpu.sync_copy(dst_hbm.at[pl.ds(base, K)], dstv)
        cpa = pltpu.async_copy(a_hbm.at[srcv], bufa, sema)
        cpb = pltpu.async_copy(b_hbm.at[dstv], bufb, semb)
        cpc = pltpu.async_copy(c_hbm.at[pl.ds(base, K)], bufc, semc)
        cpa.wait()
        cpb.wait()
        cpc.wait()

        def row(i, c2):
            for j in range(D // LANES):
                s = pl.ds(j * LANES, LANES)
                bufc[i, s] = jnp.maximum(bufa[i, s] + bufb[i, s] + bufc[i, s],
                                         0.0)
            return c2

        lax.fori_loop(0, K, row, 0)
        pltpu.sync_copy(bufc, aggs.at[dstv], add=True)
        return carry

    lax.fori_loop(0, NCH, chunk, 0)
    plsc.subcore_barrier()
    for t in range((ZCH + NS - 1) // NS):
        c = sid + NS * t
        @pl.when(c < ZCH)
        def _():
            pltpu.sync_copy(aggs.at[pl.ds(c * K, K)],
                            out_hbm.at[cid, pl.ds(c * K, K)])


def kernel(coords, edge_index, edge_features,
           W_in, b_in, W_msg, b_msg, W_upd, b_upd, W_out, b_out):
    coords8 = jnp.pad(coords, ((0, 0), (0, 5)))
    win8 = jnp.pad(W_in, ((0, 5), (0, 0)))
    W1 = W_msg[:D]
    W2 = W_msg[D:2 * D]
    W3 = W_msg[2 * D:]
    U1 = W_upd[:D]
    U2 = W_upd[D:]
    src = edge_index[0].astype(jnp.int32)
    dst = edge_index[1].astype(jnp.int32)
    b_in2 = b_in.reshape(1, D)
    b_msg2 = b_msg.reshape(1, D)
    b_upd2 = b_upd.reshape(1, D)
    b_out2 = b_out.reshape(1, NL)

    full = lambda i: (0, 0)
    rows = lambda i: (i, 0)

    h, A, B = pl.pallas_call(
        _tc_nodes,
        grid=(N // NB,),
        in_specs=[
            pl.BlockSpec((NB, 8), rows),
            pl.BlockSpec((8, D), full),
            pl.BlockSpec((1, D), full),
            pl.BlockSpec((D, D), full),
            pl.BlockSpec((D, D), full),
            pl.BlockSpec((1, D), full),
        ],
        out_specs=[
            pl.BlockSpec((NB, D), rows),
            pl.BlockSpec((NB, D), rows),
            pl.BlockSpec((NB, D), rows),
        ],
        out_shape=[
            jax.ShapeDtypeStruct((N, D), jnp.float32),
            jax.ShapeDtypeStruct((N, D), jnp.float32),
            jax.ShapeDtypeStruct((N, D), jnp.float32),
        ],
    )(coords8, win8, b_in2, W1, W2, b_msg2)

    C = pl.pallas_call(
        _tc_edgefeat,
        grid=(E // EB,),
        in_specs=[
            pl.BlockSpec((EB, DE), rows),
            pl.BlockSpec((DE, D), full),
        ],
        out_specs=pl.BlockSpec((EB, D), rows),
        out_shape=jax.ShapeDtypeStruct((E, D), jnp.float32),
    )(edge_features, W3)

    mesh = plsc.VectorSubcoreMesh(core_axis_name="c", subcore_axis_name="s",
                                  num_cores=NC, num_subcores=NS)
    agg2 = pl.kernel(
        _sc_edges,
        out_type=jax.ShapeDtypeStruct((NC, N, D), jnp.float32),
        mesh=mesh,
        scratch_types=[
            pltpu.VMEM((K,), jnp.int32),
            pltpu.VMEM((K,), jnp.int32),
            pltpu.VMEM((K, D), jnp.float32),
            pltpu.VMEM((K, D), jnp.float32),
            pltpu.VMEM((K, D), jnp.float32),
            pltpu.VMEM_SHARED((N, D), jnp.float32),
            pltpu.SemaphoreType.DMA,
            pltpu.SemaphoreType.DMA,
            pltpu.SemaphoreType.DMA,
        ],
    )(A, B, C, src, dst)

    logits = pl.pallas_call(
        _tc_out,
        grid=(N // NB,),
        in_specs=[
            pl.BlockSpec((NB, D), rows),
            pl.BlockSpec((NB, D), rows),
            pl.BlockSpec((NB, D), rows),
            pl.BlockSpec((D, D), full),
            pl.BlockSpec((D, D), full),
            pl.BlockSpec((1, D), full),
            pl.BlockSpec((D, NL), full),
            pl.BlockSpec((1, NL), full),
        ],
        out_specs=pl.BlockSpec((NB, NL), rows),
        out_shape=jax.ShapeDtypeStruct((N, NL), jnp.float32),
    )(h, agg2[0], agg2[1], U1, U2, b_upd2, W_out, b_out2)

    return logits


# speedup vs baseline: 3.6314x; 1.1397x over previous
import jax
import jax.numpy as jnp
from jax import lax
from jax.experimental import pallas as pl
from jax.experimental.pallas import tpu as pltpu
from jax.experimental.pallas import tpu_sc as plsc

N = 10000
E = 320000
D = 128
DE = 16
NL = 21
TEMP = 0.1

NC = 2
NS = 16
NW = NC * NS
ET = E // NW
K = 40
NCH = ET // K
ZCH = N // K

LANES = 16
NB = 1000
EB = 2000


def _tc_nodes(coords_ref, win_ref, bin_ref, w1_ref, w2_ref, bmsg_ref,
              h_ref, a_ref, b_ref):
    h = jnp.maximum(
        jnp.dot(coords_ref[...], win_ref[...],
                preferred_element_type=jnp.float32) + bin_ref[...], 0.0)
    h_ref[...] = h
    a_ref[...] = jnp.dot(h, w1_ref[...], preferred_element_type=jnp.float32)
    b_ref[...] = (jnp.dot(h, w2_ref[...], preferred_element_type=jnp.float32)
                  + bmsg_ref[...])


def _tc_edgefeat(ef_ref, w3_ref, c_ref):
    c_ref[...] = jnp.dot(ef_ref[...], w3_ref[...],
                         preferred_element_type=jnp.float32)


def _tc_out(h_ref, a0_ref, a1_ref, u1_ref, u2_ref, bupd_ref,
            wout_ref, bout_ref, out_ref):
    agg = a0_ref[...] + a1_ref[...]
    h2 = jnp.maximum(
        jnp.dot(h_ref[...], u1_ref[...], preferred_element_type=jnp.float32)
        + jnp.dot(agg, u2_ref[...], preferred_element_type=jnp.float32)
        + bupd_ref[...], 0.0)
    out_ref[...] = (jnp.dot(h2, wout_ref[...],
                            preferred_element_type=jnp.float32)
                    + bout_ref[...]) * (1.0 / TEMP)


def _sc_edges(a_hbm, b_hbm, c_hbm, src_hbm, dst_hbm, out_hbm,
              srcv0, dstv0, bufa0, bufb0, bufc0,
              srcv1, dstv1, bufa1, bufb1, bufc1,
              aggs, sema0, semb0, semc0, sema1, semb1, semc1):
    cid = lax.axis_index("c")
    sid = lax.axis_index("s")
    wid = cid * NS + sid
    tile_base = wid * ET

    srcv = (srcv0, srcv1)
    dstv = (dstv0, dstv1)
    bufa = (bufa0, bufa1)
    bufb = (bufb0, bufb1)
    bufc = (bufc0, bufc1)
    sema = (sema0, sema1)
    semb = (semb0, semb1)
    semc = (semc0, semc1)

    zero16 = jnp.zeros((LANES,), jnp.float32)

    def zfill(i, carry):
        for j in range(D // LANES):
            bufc0[i, pl.ds(j * LANES, LANES)] = zero16
        return carry

    lax.fori_loop(0, K, zfill, 0)
    for t in range((ZCH + NS - 1) // NS):
        c = sid + NS * t
        @pl.when(c < ZCH)
        def _():
            pltpu.sync_copy(bufc0, aggs.at[pl.ds(c * K, K)])
    plsc.subcore_barrier()

    def fire(ch, b):
        base = tile_base + ch * K
        pltpu.sync_copy(src_hbm.at[pl.ds(base, K)], srcv[b])
        pltpu.sync_copy(dst_hbm.at[pl.ds(base, K)], dstv[b])
        pltpu.async_copy(a_hbm.at[srcv[b]], bufa[b], sema[b])
        pltpu.async_copy(b_hbm.at[dstv[b]], bufb[b], semb[b])
        pltpu.async_copy(c_hbm.at[pl.ds(base, K)], bufc[b], semc[b])

    def drain(b):
        pltpu.make_async_copy(a_hbm.at[pl.ds(0, K)], bufa[b], sema[b]).wait()
        pltpu.make_async_copy(b_hbm.at[pl.ds(0, K)], bufb[b], semb[b]).wait()
        pltpu.make_async_copy(c_hbm.at[pl.ds(0, K)], bufc[b], semc[b]).wait()

    def consume(b):
        def row(i, c2):
            for j in range(D // LANES):
                s = pl.ds(j * LANES, LANES)
                bufc[b][i, s] = jnp.maximum(
                    bufa[b][i, s] + bufb[b][i, s] + bufc[b][i, s], 0.0)
            return c2

        lax.fori_loop(0, K, row, 0)
        pltpu.sync_copy(bufc[b], aggs.at[dstv[b]], add=True)

    fire(0, 0)

    def pair(i, carry):
        for b in range(2):
            s = 2 * i + b
            @pl.when(s + 1 < NCH)
            def _():
                fire(s + 1, 1 - b)
            drain(b)
            consume(b)
        return carry

    lax.fori_loop(0, NCH // 2, pair, 0)
    plsc.subcore_barrier()
    for t in range((ZCH + NS - 1) // NS):
        c = sid + NS * t
        @pl.when(c < ZCH)
        def _():
            pltpu.sync_copy(aggs.at[pl.ds(c * K, K)],
                            out_hbm.at[cid, pl.ds(c * K, K)])


def kernel(coords, edge_index, edge_features,
           W_in, b_in, W_msg, b_msg, W_upd, b_upd, W_out, b_out):
    coords8 = jnp.pad(coords, ((0, 0), (0, 5)))
    win8 = jnp.pad(W_in, ((0, 5), (0, 0)))
    W1 = W_msg[:D]
    W2 = W_msg[D:2 * D]
    W3 = W_msg[2 * D:]
    U1 = W_upd[:D]
    U2 = W_upd[D:]
    src = edge_index[0].astype(jnp.int32)
    dst = edge_index[1].astype(jnp.int32)
    b_in2 = b_in.reshape(1, D)
    b_msg2 = b_msg.reshape(1, D)
    b_upd2 = b_upd.reshape(1, D)
    b_out2 = b_out.reshape(1, NL)

    full = lambda i: (0, 0)
    rows = lambda i: (i, 0)

    h, A, B = pl.pallas_call(
        _tc_nodes,
        grid=(N // NB,),
        in_specs=[
            pl.BlockSpec((NB, 8), rows),
            pl.BlockSpec((8, D), full),
            pl.BlockSpec((1, D), full),
            pl.BlockSpec((D, D), full),
            pl.BlockSpec((D, D), full),
            pl.BlockSpec((1, D), full),
        ],
        out_specs=[
            pl.BlockSpec((NB, D), rows),
            pl.BlockSpec((NB, D), rows),
            pl.BlockSpec((NB, D), rows),
        ],
        out_shape=[
            jax.ShapeDtypeStruct((N, D), jnp.float32),
            jax.ShapeDtypeStruct((N, D), jnp.float32),
            jax.ShapeDtypeStruct((N, D), jnp.float32),
        ],
    )(coords8, win8, b_in2, W1, W2, b_msg2)

    C = pl.pallas_call(
        _tc_edgefeat,
        grid=(E // EB,),
        in_specs=[
            pl.BlockSpec((EB, DE), rows),
            pl.BlockSpec((DE, D), full),
        ],
        out_specs=pl.BlockSpec((EB, D), rows),
        out_shape=jax.ShapeDtypeStruct((E, D), jnp.float32),
    )(edge_features, W3)

    mesh = plsc.VectorSubcoreMesh(core_axis_name="c", subcore_axis_name="s",
                                  num_cores=NC, num_subcores=NS)
    agg2 = pl.kernel(
        _sc_edges,
        out_type=jax.ShapeDtypeStruct((NC, N, D), jnp.float32),
        mesh=mesh,
        scratch_types=[
            pltpu.VMEM((K,), jnp.int32),
            pltpu.VMEM((K,), jnp.int32),
            pltpu.VMEM((K, D), jnp.float32),
            pltpu.VMEM((K, D), jnp.float32),
            pltpu.VMEM((K, D), jnp.float32),
            pltpu.VMEM((K,), jnp.int32),
            pltpu.VMEM((K,), jnp.int32),
            pltpu.VMEM((K, D), jnp.float32),
            pltpu.VMEM((K, D), jnp.float32),
            pltpu.VMEM((K, D), jnp.float32),
            pltpu.VMEM_SHARED((N, D), jnp.float32),
            pltpu.SemaphoreType.DMA,
            pltpu.SemaphoreType.DMA,
            pltpu.SemaphoreType.DMA,
            pltpu.SemaphoreType.DMA,
            pltpu.SemaphoreType.DMA,
            pltpu.SemaphoreType.DMA,
        ],
    )(A, B, C, src, dst)

    logits = pl.pallas_call(
        _tc_out,
        grid=(N // NB,),
        in_specs=[
            pl.BlockSpec((NB, D), rows),
            pl.BlockSpec((NB, D), rows),
            pl.BlockSpec((NB, D), rows),
            pl.BlockSpec((D, D), full),
            pl.BlockSpec((D, D), full),
            pl.BlockSpec((1, D), full),
            pl.BlockSpec((D, NL), full),
            pl.BlockSpec((1, NL), full),
        ],
        out_specs=pl.BlockSpec((NB, NL), rows),
        out_shape=jax.ShapeDtypeStruct((N, NL), jnp.float32),
    )(h, agg2[0], agg2[1], U1, U2, b_upd2, W_out, b_out2)

    return logits


# speedup vs baseline: 5.9006x; 1.6249x over previous
import jax
import jax.numpy as jnp
from jax import lax
from jax.experimental import pallas as pl
from jax.experimental.pallas import tpu as pltpu
from jax.experimental.pallas import tpu_sc as plsc

N = 10000
E = 320000
D = 128
DE = 16
NL = 21
TEMP = 0.1

NC = 2
NS = 16
NW = NC * NS
ET = E // NW
K = 40
NCH = ET // K
ZCH = N // K

LANES = 16
NB = 1000
EB = 6400


def _tc_nodes(coords_ref, win_ref, bin_ref, w1_ref, w2_ref, bmsg_ref,
              h_ref, a_ref, b_ref):
    h = jnp.maximum(
        jnp.dot(coords_ref[...], win_ref[...],
                preferred_element_type=jnp.float32) + bin_ref[...], 0.0)
    h_ref[...] = h
    a_ref[...] = jnp.dot(h, w1_ref[...], preferred_element_type=jnp.float32)
    b_ref[...] = (jnp.dot(h, w2_ref[...], preferred_element_type=jnp.float32)
                  + bmsg_ref[...])


def _tc_edgefeat(eft_ref, w3_ref, c_ref):
    c_ref[...] = lax.dot_general(
        eft_ref[...], w3_ref[...],
        dimension_numbers=(((0,), (0,)), ((), ())),
        preferred_element_type=jnp.float32)


def _tc_out(h_ref, a0_ref, a1_ref, u1_ref, u2_ref, bupd_ref,
            wout_ref, bout_ref, out_ref):
    agg = a0_ref[...] + a1_ref[...]
    h2 = jnp.maximum(
        jnp.dot(h_ref[...], u1_ref[...], preferred_element_type=jnp.float32)
        + jnp.dot(agg, u2_ref[...], preferred_element_type=jnp.float32)
        + bupd_ref[...], 0.0)
    out_ref[...] = (jnp.dot(h2, wout_ref[...],
                            preferred_element_type=jnp.float32)
                    + bout_ref[...]) * (1.0 / TEMP)


def _sc_edges(a_hbm, b_hbm, c_hbm, src_hbm, dst_hbm, out_hbm,
              srcv0, srcv1, srcv2, srcv3, dstv0, dstv1, dstv2, dstv3,
              bufa0, bufb0, bufc0, bufm0, bufa1, bufb1, bufc1, bufm1,
              aggs,
              sema0, semb0, semc0, sema1, semb1, semc1,
              semi0, semi1, semi2, semi3, semsc0, semsc1):
    cid = lax.axis_index("c")
    sid = lax.axis_index("s")
    wid = cid * NS + sid
    tile_base = wid * ET

    srcv = (srcv0, srcv1, srcv2, srcv3)
    dstv = (dstv0, dstv1, dstv2, dstv3)
    bufa = (bufa0, bufa1)
    bufb = (bufb0, bufb1)
    bufc = (bufc0, bufc1)
    bufm = (bufm0, bufm1)
    sema = (sema0, sema1)
    semb = (semb0, semb1)
    semc = (semc0, semc1)
    semi = (semi0, semi1, semi2, semi3)
    semsc = (semsc0, semsc1)

    zero16 = jnp.zeros((LANES,), jnp.float32)

    def zfill(i, carry):
        for j in range(D // LANES):
            bufm0[i, pl.ds(j * LANES, LANES)] = zero16
        return carry

    lax.fori_loop(0, K, zfill, 0)
    for t in range((ZCH + NS - 1) // NS):
        c = sid + NS * t
        @pl.when(c < ZCH)
        def _():
            pltpu.sync_copy(bufm0, aggs.at[pl.ds(c * K, K)])
    plsc.subcore_barrier()

    def fire_idx(ch, q):
        base = tile_base + ch * K
        pltpu.async_copy(src_hbm.at[pl.ds(base, K)], srcv[q], semi[q])
        pltpu.async_copy(dst_hbm.at[pl.ds(base, K)], dstv[q], semi[q])

    def wait_idx(q):
        pltpu.make_async_copy(src_hbm.at[pl.ds(0, K)], srcv[q],
                              semi[q]).wait()
        pltpu.make_async_copy(src_hbm.at[pl.ds(0, K)], dstv[q],
                              semi[q]).wait()

    def fire_gathers(ch, b, q):
        base = tile_base + ch * K
        pltpu.async_copy(a_hbm.at[srcv[q]], bufa[b], sema[b])
        pltpu.async_copy(b_hbm.at[dstv[q]], bufb[b], semb[b])
        pltpu.async_copy(c_hbm.at[pl.ds(base, K)], bufc[b], semc[b])

    def drain(b):
        pltpu.make_async_copy(a_hbm.at[pl.ds(0, K)], bufa[b], sema[b]).wait()
        pltpu.make_async_copy(b_hbm.at[pl.ds(0, K)], bufb[b], semb[b]).wait()
        pltpu.make_async_copy(c_hbm.at[pl.ds(0, K)], bufc[b], semc[b]).wait()

    def wait_scatter(b):
        pltpu.make_async_copy(bufm[b], aggs.at[pl.ds(0, K)], semsc[b]).wait()

    def compute(b):
        def row(i, c2):
            for j in range(D // LANES):
                s = pl.ds(j * LANES, LANES)
                bufm[b][i, s] = jnp.maximum(
                    bufa[b][i, s] + bufb[b][i, s] + bufc[b][i, s], 0.0)
            return c2

        lax.fori_loop(0, K, row, 0)

    fire_idx(0, 0)
    wait_idx(0)
    fire_gathers(0, 0, 0)
    fire_idx(1, 1)

    def quad(i, carry):
        for u in range(4):
            s = 4 * i + u
            b = u % 2
            @pl.when(s + 1 < NCH)
            def _():
                wait_idx((u + 1) % 4)
                fire_gathers(s + 1, 1 - b, (u + 1) % 4)
            drain(b)
            @pl.when(s >= 2)
            def _():
                wait_scatter(b)
            compute(b)
            pltpu.make_async_copy(bufm[b], aggs.at[dstv[u]],
                                  semsc[b]).start(add=True)
            @pl.when(s + 2 < NCH)
            def _():
                fire_idx(s + 2, (u + 2) % 4)
        return carry

    lax.fori_loop(0, NCH // 4, quad, 0)
    for s, u in ((NCH - 2, 0), (NCH - 1, 1)):
        b = u % 2
        if s + 1 < NCH:
            wait_idx((u + 1) % 4)
            fire_gathers(s + 1, 1 - b, (u + 1) % 4)
        drain(b)
        wait_scatter(b)
        compute(b)
        pltpu.make_async_copy(bufm[b], aggs.at[dstv[u]],
                              semsc[b]).start(add=True)
    wait_scatter(0)
    wait_scatter(1)

    plsc.subcore_barrier()
    for t in range((ZCH + NS - 1) // NS):
        c = sid + NS * t
        @pl.when(c < ZCH)
        def _():
            pltpu.sync_copy(aggs.at[pl.ds(c * K, K)],
                            out_hbm.at[cid, pl.ds(c * K, K)])


def kernel(coords, edge_index, edge_features,
           W_in, b_in, W_msg, b_msg, W_upd, b_upd, W_out, b_out):
    coords8 = jnp.pad(coords, ((0, 0), (0, 5)))
    win8 = jnp.pad(W_in, ((0, 5), (0, 0)))
    W1 = W_msg[:D]
    W2 = W_msg[D:2 * D]
    W3 = W_msg[2 * D:]
    U1 = W_upd[:D]
    U2 = W_upd[D:]
    src = edge_index[0].astype(jnp.int32)
    dst = edge_index[1].astype(jnp.int32)
    b_in2 = b_in.reshape(1, D)
    b_msg2 = b_msg.reshape(1, D)
    b_upd2 = b_upd.reshape(1, D)
    b_out2 = b_out.reshape(1, NL)

    full = lambda i: (0, 0)
    rows = lambda i: (i, 0)

    h, A, B = pl.pallas_call(
        _tc_nodes,
        grid=(N // NB,),
        in_specs=[
            pl.BlockSpec((NB, 8), rows),
            pl.BlockSpec((8, D), full),
            pl.BlockSpec((1, D), full),
            pl.BlockSpec((D, D), full),
            pl.BlockSpec((D, D), full),
            pl.BlockSpec((1, D), full),
        ],
        out_specs=[
            pl.BlockSpec((NB, D), rows),
            pl.BlockSpec((NB, D), rows),
            pl.BlockSpec((NB, D), rows),
        ],
        out_shape=[
            jax.ShapeDtypeStruct((N, D), jnp.float32),
            jax.ShapeDtypeStruct((N, D), jnp.float32),
            jax.ShapeDtypeStruct((N, D), jnp.float32),
        ],
    )(coords8, win8, b_in2, W1, W2, b_msg2)

    eft = edge_features.T
    C = pl.pallas_call(
        _tc_edgefeat,
        grid=(E // EB,),
        in_specs=[
            pl.BlockSpec((DE, EB), lambda i: (0, i)),
            pl.BlockSpec((DE, D), full),
        ],
        out_specs=pl.BlockSpec((EB, D), rows),
        out_shape=jax.ShapeDtypeStruct((E, D), jnp.float32),
    )(eft, W3)

    mesh = plsc.VectorSubcoreMesh(core_axis_name="c", subcore_axis_name="s",
                                  num_cores=NC, num_subcores=NS)
    agg2 = pl.kernel(
        _sc_edges,
        out_type=jax.ShapeDtypeStruct((NC, N, D), jnp.float32),
        mesh=mesh,
        scratch_types=[
            pltpu.VMEM((K,), jnp.int32),
            pltpu.VMEM((K,), jnp.int32),
            pltpu.VMEM((K,), jnp.int32),
            pltpu.VMEM((K,), jnp.int32),
            pltpu.VMEM((K,), jnp.int32),
            pltpu.VMEM((K,), jnp.int32),
            pltpu.VMEM((K,), jnp.int32),
            pltpu.VMEM((K,), jnp.int32),
            pltpu.VMEM((K, D), jnp.float32),
            pltpu.VMEM((K, D), jnp.float32),
            pltpu.VMEM((K, D), jnp.float32),
            pltpu.VMEM((K, D), jnp.float32),
            pltpu.VMEM((K, D), jnp.float32),
            pltpu.VMEM((K, D), jnp.float32),
            pltpu.VMEM((K, D), jnp.float32),
            pltpu.VMEM((K, D), jnp.float32),
            pltpu.VMEM_SHARED((N, D), jnp.float32),
            pltpu.SemaphoreType.DMA,
            pltpu.SemaphoreType.DMA,
            pltpu.SemaphoreType.DMA,
            pltpu.SemaphoreType.DMA,
            pltpu.SemaphoreType.DMA,
            pltpu.SemaphoreType.DMA,
            pltpu.SemaphoreType.DMA,
            pltpu.SemaphoreType.DMA,
            pltpu.SemaphoreType.DMA,
            pltpu.SemaphoreType.DMA,
            pltpu.SemaphoreType.DMA,
            pltpu.SemaphoreType.DMA,
        ],
    )(A, B, C, src, dst)

    logits = pl.pallas_call(
        _tc_out,
        grid=(N // NB,),
        in_specs=[
            pl.BlockSpec((NB, D), rows),
            pl.BlockSpec((NB, D), rows),
            pl.BlockSpec((NB, D), rows),
            pl.BlockSpec((D, D), full),
            pl.BlockSpec((D, D), full),
            pl.BlockSpec((1, D), full),
            pl.BlockSpec((D, NL), full),
            pl.BlockSpec((1, NL), full),
        ],
        out_specs=pl.BlockSpec((NB, NL), rows),
        out_shape=jax.ShapeDtypeStruct((N, NL), jnp.float32),
    )(h, agg2[0], agg2[1], U1, U2, b_upd2, W_out, b_out2)

    return logits


# speedup vs baseline: 6.1397x; 1.0405x over previous
import jax
import jax.numpy as jnp
from jax import lax
from jax.experimental import pallas as pl
from jax.experimental.pallas import tpu as pltpu
from jax.experimental.pallas import tpu_sc as plsc

N = 10000
E = 320000
D = 128
DE = 16
NL = 21
TEMP = 0.1

NC = 2
NS = 16
NW = NC * NS
ET = E // NW
K = 64
KT = ET % K
NCH = ET // K
ZCH = N // K
ZR = N % K

LANES = 16
NB = 1000
EB = 6400


def _tc_nodes(coords_ref, win_ref, bin_ref, w1_ref, w2_ref, bmsg_ref,
              h_ref, a_ref, b_ref):
    h = jnp.maximum(
        jnp.dot(coords_ref[...], win_ref[...],
                preferred_element_type=jnp.float32) + bin_ref[...], 0.0)
    h_ref[...] = h
    a_ref[...] = jnp.dot(h, w1_ref[...], preferred_element_type=jnp.float32)
    b_ref[...] = (jnp.dot(h, w2_ref[...], preferred_element_type=jnp.float32)
                  + bmsg_ref[...])


def _tc_edgefeat(eft_ref, w3_ref, c_ref):
    c_ref[...] = lax.dot_general(
        eft_ref[...], w3_ref[...],
        dimension_numbers=(((0,), (0,)), ((), ())),
        preferred_element_type=jnp.float32)


def _tc_out(h_ref, a0_ref, a1_ref, u1_ref, u2_ref, bupd_ref,
            wout_ref, bout_ref, out_ref):
    agg = a0_ref[...] + a1_ref[...]
    h2 = jnp.maximum(
        jnp.dot(h_ref[...], u1_ref[...], preferred_element_type=jnp.float32)
        + jnp.dot(agg, u2_ref[...], preferred_element_type=jnp.float32)
        + bupd_ref[...], 0.0)
    out_ref[...] = (jnp.dot(h2, wout_ref[...],
                            preferred_element_type=jnp.float32)
                    + bout_ref[...]) * (1.0 / TEMP)


def _sc_edges(a_hbm, b_hbm, c_hbm, src_hbm, dst_hbm, out_hbm,
              srcv0, dstv0, bufa0, bufb0, bufc0,
              srcv1, dstv1, bufa1, bufb1, bufc1,
              srcvt, dstvt,
              aggs, sema0, semb0, semc0, sema1, semb1, semc1, semi0, semi1):
    cid = lax.axis_index("c")
    sid = lax.axis_index("s")
    wid = cid * NS + sid
    tile_base = wid * ET

    srcv = (srcv0, srcv1)
    dstv = (dstv0, dstv1)
    bufa = (bufa0, bufa1)
    bufb = (bufb0, bufb1)
    bufc = (bufc0, bufc1)
    sema = (sema0, sema1)
    semb = (semb0, semb1)
    semc = (semc0, semc1)
    semi = (semi0, semi1)

    zero16 = jnp.zeros((LANES,), jnp.float32)

    def zfill(i, carry):
        for j in range(D // LANES):
            bufc0[i, pl.ds(j * LANES, LANES)] = zero16
        return carry

    lax.fori_loop(0, K, zfill, 0)
    for t in range((ZCH + NS - 1) // NS):
        c = sid + NS * t
        @pl.when(c < ZCH)
        def _():
            pltpu.sync_copy(bufc0, aggs.at[pl.ds(c * K, K)])
    @pl.when(sid == 0)
    def _():
        pltpu.sync_copy(bufc0.at[pl.ds(0, ZR)], aggs.at[pl.ds(ZCH * K, ZR)])
    plsc.subcore_barrier()

    def fire_idx(ch, b):
        base = tile_base + ch * K
        pltpu.async_copy(src_hbm.at[pl.ds(base, K)], srcv[b], semi[b])
        pltpu.async_copy(dst_hbm.at[pl.ds(base, K)], dstv[b], semi[b])

    def wait_idx(b):
        pltpu.make_async_copy(src_hbm.at[pl.ds(0, K)], srcv[b], semi[b]).wait()
        pltpu.make_async_copy(dst_hbm.at[pl.ds(0, K)], dstv[b], semi[b]).wait()

    def fire_gathers(ch, b):
        base = tile_base + ch * K
        pltpu.async_copy(a_hbm.at[srcv[b]], bufa[b], sema[b])
        pltpu.async_copy(b_hbm.at[dstv[b]], bufb[b], semb[b])
        pltpu.async_copy(c_hbm.at[pl.ds(base, K)], bufc[b], semc[b])

    def drain(b):
        pltpu.make_async_copy(a_hbm.at[pl.ds(0, K)], bufa[b], sema[b]).wait()
        pltpu.make_async_copy(b_hbm.at[pl.ds(0, K)], bufb[b], semb[b]).wait()
        pltpu.make_async_copy(c_hbm.at[pl.ds(0, K)], bufc[b], semc[b]).wait()

    def consume(b):
        def row(i, c2):
            for j in range(D // LANES):
                s = pl.ds(j * LANES, LANES)
                bufc[b][i, s] = jnp.maximum(
                    bufa[b][i, s] + bufb[b][i, s] + bufc[b][i, s], 0.0)
            return c2

        lax.fori_loop(0, K, row, 0)
        pltpu.sync_copy(bufc[b], aggs.at[dstv[b]], add=True)

    fire_idx(0, 0)
    wait_idx(0)
    fire_gathers(0, 0)
    fire_idx(1, 1)

    def pair(i, carry):
        for b in range(2):
            s = 2 * i + b
            nb = 1 - b
            @pl.when(s + 1 < NCH)
            def _():
                wait_idx(nb)
                fire_gathers(s + 1, nb)
            drain(b)
            consume(b)
            @pl.when(s + 2 < NCH)
            def _():
                fire_idx(s + 2, b)
        return carry

    lax.fori_loop(0, NCH // 2, pair, 0)

    tbase = tile_base + NCH * K
    pltpu.sync_copy(src_hbm.at[pl.ds(tbase, KT)], srcvt)
    pltpu.sync_copy(dst_hbm.at[pl.ds(tbase, KT)], dstvt)
    cpa = pltpu.async_copy(a_hbm.at[srcvt], bufa0.at[pl.ds(0, KT)], sema0)
    cpb = pltpu.async_copy(b_hbm.at[dstvt], bufb0.at[pl.ds(0, KT)], semb0)
    cpc = pltpu.async_copy(c_hbm.at[pl.ds(tbase, KT)],
                           bufc0.at[pl.ds(0, KT)], semc0)
    cpa.wait()
    cpb.wait()
    cpc.wait()

    def trow(i, c2):
        for j in range(D // LANES):
            s = pl.ds(j * LANES, LANES)
            bufc0[i, s] = jnp.maximum(bufa0[i, s] + bufb0[i, s] + bufc0[i, s],
                                      0.0)
        return c2

    lax.fori_loop(0, KT, trow, 0)
    pltpu.sync_copy(bufc0.at[pl.ds(0, KT)], aggs.at[dstvt], add=True)

    plsc.subcore_barrier()
    for t in range((ZCH + NS - 1) // NS):
        c = sid + NS * t
        @pl.when(c < ZCH)
        def _():
            pltpu.sync_copy(aggs.at[pl.ds(c * K, K)],
                            out_hbm.at[cid, pl.ds(c * K, K)])
    @pl.when(sid == 0)
    def _():
        pltpu.sync_copy(aggs.at[pl.ds(ZCH * K, ZR)],
                        out_hbm.at[cid, pl.ds(ZCH * K, ZR)])


def kernel(coords, edge_index, edge_features,
           W_in, b_in, W_msg, b_msg, W_upd, b_upd, W_out, b_out):
    coords8 = jnp.pad(coords, ((0, 0), (0, 5)))
    win8 = jnp.pad(W_in, ((0, 5), (0, 0)))
    W1 = W_msg[:D]
    W2 = W_msg[D:2 * D]
    W3 = W_msg[2 * D:]
    U1 = W_upd[:D]
    U2 = W_upd[D:]
    src = edge_index[0].astype(jnp.int32)
    dst = edge_index[1].astype(jnp.int32)
    b_in2 = b_in.reshape(1, D)
    b_msg2 = b_msg.reshape(1, D)
    b_upd2 = b_upd.reshape(1, D)
    b_out2 = b_out.reshape(1, NL)

    full = lambda i: (0, 0)
    rows = lambda i: (i, 0)

    h, A, B = pl.pallas_call(
        _tc_nodes,
        grid=(N // NB,),
        in_specs=[
            pl.BlockSpec((NB, 8), rows),
            pl.BlockSpec((8, D), full),
            pl.BlockSpec((1, D), full),
            pl.BlockSpec((D, D), full),
            pl.BlockSpec((D, D), full),
            pl.BlockSpec((1, D), full),
        ],
        out_specs=[
            pl.BlockSpec((NB, D), rows),
            pl.BlockSpec((NB, D), rows),
            pl.BlockSpec((NB, D), rows),
        ],
        out_shape=[
            jax.ShapeDtypeStruct((N, D), jnp.float32),
            jax.ShapeDtypeStruct((N, D), jnp.float32),
            jax.ShapeDtypeStruct((N, D), jnp.float32),
        ],
    )(coords8, win8, b_in2, W1, W2, b_msg2)

    eft = edge_features.T
    C = pl.pallas_call(
        _tc_edgefeat,
        grid=(E // EB,),
        in_specs=[
            pl.BlockSpec((DE, EB), lambda i: (0, i)),
            pl.BlockSpec((DE, D), full),
        ],
        out_specs=pl.BlockSpec((EB, D), rows),
        out_shape=jax.ShapeDtypeStruct((E, D), jnp.float32),
    )(eft, W3)

    mesh = plsc.VectorSubcoreMesh(core_axis_name="c", subcore_axis_name="s",
                                  num_cores=NC, num_subcores=NS)
    agg2 = pl.kernel(
        _sc_edges,
        out_type=jax.ShapeDtypeStruct((NC, N, D), jnp.float32),
        mesh=mesh,
        scratch_types=[
            pltpu.VMEM((K,), jnp.int32),
            pltpu.VMEM((K,), jnp.int32),
            pltpu.VMEM((K, D), jnp.float32),
            pltpu.VMEM((K, D), jnp.float32),
            pltpu.VMEM((K, D), jnp.float32),
            pltpu.VMEM((K,), jnp.int32),
            pltpu.VMEM((K,), jnp.int32),
            pltpu.VMEM((K, D), jnp.float32),
            pltpu.VMEM((K, D), jnp.float32),
            pltpu.VMEM((K, D), jnp.float32),
            pltpu.VMEM((KT,), jnp.int32),
            pltpu.VMEM((KT,), jnp.int32),
            pltpu.VMEM_SHARED((N, D), jnp.float32),
            pltpu.SemaphoreType.DMA,
            pltpu.SemaphoreType.DMA,
            pltpu.SemaphoreType.DMA,
            pltpu.SemaphoreType.DMA,
            pltpu.SemaphoreType.DMA,
            pltpu.SemaphoreType.DMA,
            pltpu.SemaphoreType.DMA,
            pltpu.SemaphoreType.DMA,
        ],
    )(A, B, C, src, dst)

    logits = pl.pallas_call(
        _tc_out,
        grid=(N // NB,),
        in_specs=[
            pl.BlockSpec((NB, D), rows),
            pl.BlockSpec((NB, D), rows),
            pl.BlockSpec((NB, D), rows),
            pl.BlockSpec((D, D), full),
            pl.BlockSpec((D, D), full),
            pl.BlockSpec((1, D), full),
            pl.BlockSpec((D, NL), full),
            pl.BlockSpec((1, NL), full),
        ],
        out_specs=pl.BlockSpec((NB, NL), rows),
        out_shape=jax.ShapeDtypeStruct((N, NL), jnp.float32),
    )(h, agg2[0], agg2[1], U1, U2, b_upd2, W_out, b_out2)

    return logits
